# baseline (speedup 1.0000x reference)
import functools

import jax
import jax.numpy as jnp
from jax import lax
from jax.experimental import pallas as pl
from jax.experimental.pallas import tpu as pltpu
from jax.experimental.pallas import tpu_sc as plsc

BATCH = 16384
HIST = 200
HIDDEN = 32
TOTAL = BATCH * HIST

NUM_CORES = 2
NUM_SUBCORES = 16
NW = NUM_CORES * NUM_SUBCORES
PER_W = TOTAL // NW
CHUNK = 512
NCHUNK = PER_W // CHUNK
NPAIR = NCHUNK // 2
NGRP = 3
NSLOT = 2 * NGRP

_mesh = plsc.VectorSubcoreMesh(core_axis_name="c", subcore_axis_name="s")


@functools.partial(
    pl.kernel,
    out_type=jax.ShapeDtypeStruct((TOTAL, HIDDEN), jnp.float32),
    mesh=_mesh,
    scratch_types=[
        pltpu.VMEM((NSLOT, CHUNK), jnp.int32),
        pltpu.VMEM((NSLOT, CHUNK, HIDDEN), jnp.float32),
        pltpu.SemaphoreType.DMA((NSLOT,)),
        pltpu.SemaphoreType.DMA((NGRP,)),
        pltpu.SemaphoreType.DMA((NSLOT,)),
    ],
    compiler_params=pltpu.CompilerParams(use_tc_tiling_on_sc=False),
)
def _emb_lookup(x_hbm, tab_hbm, out_hbm, idx_v, rows_v, isem, gsem, osem):
    wid = lax.axis_index("s") * NUM_CORES + lax.axis_index("c")
    base = wid * PER_W

    def xs(j):
        return x_hbm.at[pl.ds(base + j * CHUNK, CHUNK)]

    def outs(j):
        return out_hbm.at[pl.ds(base + j * CHUNK, CHUNK)]

    def istart(j, b):
        pltpu.async_copy(xs(j), idx_v.at[b], isem.at[b])

    def iwait(j, b):
        pltpu.make_async_copy(xs(j), idx_v.at[b], isem.at[b]).wait()

    def gstart(b, e):
        pltpu.async_copy(tab_hbm.at[idx_v.at[b]], rows_v.at[b], gsem.at[e])

    def gwait(b, e):
        pltpu.make_async_copy(tab_hbm.at[idx_v.at[b]], rows_v.at[b],
                              gsem.at[e]).wait()

    def ostart(j, b):
        pltpu.async_copy(rows_v.at[b], outs(j), osem.at[b])

    def owait(j, b):
        pltpu.make_async_copy(rows_v.at[b], outs(j), osem.at[b]).wait()

    def fire(p, e, wait_rows):
        j0 = 2 * p
        s0, s1 = 2 * e, 2 * e + 1
        iwait(j0, s0)
        iwait(j0 + 1, s1)
        if wait_rows:
            owait(j0 - NSLOT, s0)
            owait(j0 - NSLOT + 1, s1)
        gstart(s0, e)
        gstart(s1, e)

    def drain(p, e, refill):
        j0 = 2 * p
        s0, s1 = 2 * e, 2 * e + 1
        gwait(s0, e)
        gwait(s1, e)
        ostart(j0, s0)
        ostart(j0 + 1, s1)
        if refill:
            istart(j0 + NSLOT, s0)
            istart(j0 + NSLOT + 1, s1)

    for j in range(NSLOT):
        istart(j, j)
    for e in range(NGRP):
        fire(e, e, False)

    def body(k, carry):
        for e in range(NGRP):
            p = NGRP * k + e
            drain(p, e, True)
            fire(p + NGRP, e, True)
        return carry

    nsteady = (NPAIR - NGRP - 1) // NGRP
    lax.fori_loop(0, nsteady, body, 0)

    for p in range(NGRP * nsteady, NPAIR):
        e = p % NGRP
        drain(p, e, 2 * p + NSLOT + 1 <= NCHUNK - 1)
        if p + NGRP <= NPAIR - 1:
            fire(p + NGRP, e, True)
    for j in range(NCHUNK - NSLOT, NCHUNK):
        owait(j, j % NSLOT)


def kernel(x, table):
    flat = x.reshape(TOTAL)
    out = _emb_lookup(flat, table)
    return out.reshape(BATCH, HIST, HIDDEN)
